# baseline (speedup 1.0000x reference)
import functools

import jax
import jax.numpy as jnp
from jax.experimental import pallas as pl
from jax.experimental.pallas import tpu as pltpu

BM = 512
BK = 1024


def _mm_body(x1_ref, x2_ref, w1_ref, w2_ref, n1_ref, n2_ref,
             h1_ref, h2_ref, xw1, xw2, *, bk):
    m = pl.program_id(0)
    k = pl.program_id(1)

    @pl.when((m == 0) & (k == 0))
    def _():
        xw1[...] = jnp.dot(x1_ref[...], w1_ref[...],
                           preferred_element_type=jnp.float32)
        xw2[...] = jnp.dot(x2_ref[...], w2_ref[...],
                           preferred_element_type=jnp.float32)

    @pl.when(k == 0)
    def _():
        h1_ref[...] = jnp.zeros_like(h1_ref)
        h2_ref[...] = jnp.zeros_like(h2_ref)

    a1 = jnp.dot(n1_ref[...], xw1[pl.ds(k * bk, bk), :],
                 preferred_element_type=jnp.float32,
                 precision=jax.lax.Precision.DEFAULT)
    a2 = jnp.dot(n2_ref[...], xw2[pl.ds(k * bk, bk), :],
                 preferred_element_type=jnp.float32,
                 precision=jax.lax.Precision.DEFAULT)
    h1_ref[...] += a1
    h2_ref[...] += a2


def _fin_body(h1_ref, h2_ref, o_ref):
    h1 = h1_ref[...]
    h2 = h2_ref[...]
    w1 = jax.nn.sigmoid(jnp.sum(h1, axis=0, keepdims=True))
    w2 = jax.nn.sigmoid(jnp.sum(h2, axis=0, keepdims=True))
    dn = (((1,), (1,)), ((), ()))
    r1 = jax.nn.sigmoid(jax.lax.dot_general(
        h1, w1, dn, preferred_element_type=jnp.float32))
    r2 = jax.nn.sigmoid(jax.lax.dot_general(
        h2, w2, dn, preferred_element_type=jnp.float32))
    o_ref[...] = jax.nn.sigmoid((r1 * h1 + r2 * h2) * 0.5)


def kernel(x_1, x_2, neighborhood_1, neighborhood_2, W1, W2):
    n_m, n_k = neighborhood_1.shape
    _, n_l = neighborhood_2.shape
    f = W1.shape[1]
    bm, bk = min(BM, n_m), min(BK, n_k)

    h1, h2 = pl.pallas_call(
        functools.partial(_mm_body, bk=bk),
        grid=(n_m // bm, n_k // bk),
        in_specs=[
            pl.BlockSpec((n_k, x_1.shape[1]), lambda m, k: (0, 0)),
            pl.BlockSpec((n_l, x_2.shape[1]), lambda m, k: (0, 0)),
            pl.BlockSpec(W1.shape, lambda m, k: (0, 0)),
            pl.BlockSpec(W2.shape, lambda m, k: (0, 0)),
            pl.BlockSpec((bm, bk), lambda m, k: (m, k)),
            pl.BlockSpec((bm, bk), lambda m, k: (m, k)),
        ],
        out_specs=[
            pl.BlockSpec((bm, f), lambda m, k: (m, 0)),
            pl.BlockSpec((bm, f), lambda m, k: (m, 0)),
        ],
        out_shape=[
            jax.ShapeDtypeStruct((n_m, f), jnp.float32),
            jax.ShapeDtypeStruct((n_m, f), jnp.float32),
        ],
        scratch_shapes=[
            pltpu.VMEM((n_k, f), jnp.float32),
            pltpu.VMEM((n_l, f), jnp.float32),
        ],
        compiler_params=pltpu.CompilerParams(
            dimension_semantics=("parallel", "arbitrary"),
        ),
    )(x_1, x_2, W1, W2, neighborhood_1, neighborhood_2)

    out = pl.pallas_call(
        _fin_body,
        out_shape=jax.ShapeDtypeStruct((n_m, f), jnp.float32),
    )(h1, h2)
    return out


# speedup vs baseline: 1.2132x; 1.2132x over previous
import functools

import jax
import jax.numpy as jnp
from jax.experimental import pallas as pl
from jax.experimental.pallas import tpu as pltpu

BM = 128


def _mm_body(x1_ref, x2_ref, w1_ref, w2_ref, n1_ref, n2_ref,
             h1_ref, h2_ref, xw1, xw2):
    m = pl.program_id(0)

    @pl.when(m == 0)
    def _():
        xw1[...] = jnp.dot(x1_ref[...], w1_ref[...],
                           preferred_element_type=jnp.float32)
        xw2[...] = jnp.dot(x2_ref[...], w2_ref[...],
                           preferred_element_type=jnp.float32)

    h1_ref[...] = jnp.dot(n1_ref[...], xw1[...],
                          preferred_element_type=jnp.float32)
    h2_ref[...] = jnp.dot(n2_ref[...], xw2[...],
                          preferred_element_type=jnp.float32)


def _fin_body(h1_ref, h2_ref, o_ref):
    h1 = h1_ref[...]
    h2 = h2_ref[...]
    w1 = jax.nn.sigmoid(jnp.sum(h1, axis=0, keepdims=True))
    w2 = jax.nn.sigmoid(jnp.sum(h2, axis=0, keepdims=True))
    dn = (((1,), (1,)), ((), ()))
    r1 = jax.nn.sigmoid(jax.lax.dot_general(
        h1, w1, dn, preferred_element_type=jnp.float32))
    r2 = jax.nn.sigmoid(jax.lax.dot_general(
        h2, w2, dn, preferred_element_type=jnp.float32))
    o_ref[...] = jax.nn.sigmoid((r1 * h1 + r2 * h2) * 0.5)


def kernel(x_1, x_2, neighborhood_1, neighborhood_2, W1, W2):
    n_m, n_k = neighborhood_1.shape
    _, n_l = neighborhood_2.shape
    f = W1.shape[1]
    bm = min(BM, n_m)

    h1, h2 = pl.pallas_call(
        _mm_body,
        grid=(n_m // bm,),
        in_specs=[
            pl.BlockSpec((n_k, x_1.shape[1]), lambda m: (0, 0)),
            pl.BlockSpec((n_l, x_2.shape[1]), lambda m: (0, 0)),
            pl.BlockSpec(W1.shape, lambda m: (0, 0)),
            pl.BlockSpec(W2.shape, lambda m: (0, 0)),
            pl.BlockSpec((bm, n_k), lambda m: (m, 0)),
            pl.BlockSpec((bm, n_l), lambda m: (m, 0)),
        ],
        out_specs=[
            pl.BlockSpec((bm, f), lambda m: (m, 0)),
            pl.BlockSpec((bm, f), lambda m: (m, 0)),
        ],
        out_shape=[
            jax.ShapeDtypeStruct((n_m, f), jnp.float32),
            jax.ShapeDtypeStruct((n_m, f), jnp.float32),
        ],
        scratch_shapes=[
            pltpu.VMEM((n_k, f), jnp.float32),
            pltpu.VMEM((n_l, f), jnp.float32),
        ],
        compiler_params=pltpu.CompilerParams(
            dimension_semantics=("arbitrary",),
        ),
    )(x_1, x_2, W1, W2, neighborhood_1, neighborhood_2)

    out = pl.pallas_call(
        _fin_body,
        out_shape=jax.ShapeDtypeStruct((n_m, f), jnp.float32),
    )(h1, h2)
    return out


# speedup vs baseline: 1.2195x; 1.0052x over previous
import functools

import jax
import jax.numpy as jnp
from jax.experimental import pallas as pl
from jax.experimental.pallas import tpu as pltpu

BM = 128


def _mm_body(x1_ref, x2_ref, w1_ref, w2_ref, n1_ref, n2_ref,
             h1_ref, h2_ref, xw1, xw2):
    m = pl.program_id(0)

    @pl.when(m == 0)
    def _():
        xw1[...] = jnp.dot(x1_ref[...], w1_ref[...],
                           preferred_element_type=jnp.float32)
        xw2[...] = jnp.dot(x2_ref[...], w2_ref[...],
                           preferred_element_type=jnp.float32)

    h1_ref[...] = jnp.dot(n1_ref[...], xw1[...],
                          preferred_element_type=jnp.float32)
    h2_ref[...] = jnp.dot(n2_ref[...], xw2[...],
                          preferred_element_type=jnp.float32)


def _sigmoid(x):
    return 0.5 + 0.5 * jnp.tanh(0.5 * x)


def _fin_body(h1_ref, h2_ref, o_ref):
    h1 = h1_ref[...]
    h2 = h2_ref[...]
    w1 = _sigmoid(jnp.sum(h1, axis=0, keepdims=True))
    w2 = _sigmoid(jnp.sum(h2, axis=0, keepdims=True))
    dn = (((1,), (1,)), ((), ()))
    r1 = _sigmoid(jax.lax.dot_general(
        h1, w1, dn, preferred_element_type=jnp.float32))
    r2 = _sigmoid(jax.lax.dot_general(
        h2, w2, dn, preferred_element_type=jnp.float32))
    o_ref[...] = _sigmoid((r1 * h1 + r2 * h2) * 0.5)


def kernel(x_1, x_2, neighborhood_1, neighborhood_2, W1, W2):
    n_m, n_k = neighborhood_1.shape
    _, n_l = neighborhood_2.shape
    f = W1.shape[1]
    bm = min(BM, n_m)

    h1, h2 = pl.pallas_call(
        _mm_body,
        grid=(n_m // bm,),
        in_specs=[
            pl.BlockSpec((n_k, x_1.shape[1]), lambda m: (0, 0)),
            pl.BlockSpec((n_l, x_2.shape[1]), lambda m: (0, 0)),
            pl.BlockSpec(W1.shape, lambda m: (0, 0)),
            pl.BlockSpec(W2.shape, lambda m: (0, 0)),
            pl.BlockSpec((bm, n_k), lambda m: (m, 0)),
            pl.BlockSpec((bm, n_l), lambda m: (m, 0)),
        ],
        out_specs=[
            pl.BlockSpec((bm, f), lambda m: (m, 0)),
            pl.BlockSpec((bm, f), lambda m: (m, 0)),
        ],
        out_shape=[
            jax.ShapeDtypeStruct((n_m, f), jnp.float32),
            jax.ShapeDtypeStruct((n_m, f), jnp.float32),
        ],
        scratch_shapes=[
            pltpu.VMEM((n_k, f), jnp.float32),
            pltpu.VMEM((n_l, f), jnp.float32),
        ],
        compiler_params=pltpu.CompilerParams(
            dimension_semantics=("arbitrary",),
        ),
    )(x_1, x_2, W1, W2, neighborhood_1, neighborhood_2)

    out = pl.pallas_call(
        _fin_body,
        out_shape=jax.ShapeDtypeStruct((n_m, f), jnp.float32),
    )(h1, h2)
    return out
